# Initial kernel scaffold; baseline (speedup 1.0000x reference)
import jax
import jax.numpy as jnp
from jax.experimental import pallas as pl


def kernel(x, emb, W1, b1, W2, b2):
    raise NotImplementedError("write your pallas kernel here")


# speedup vs baseline: 13.2531x; 13.2531x over previous
import functools

import jax
import jax.numpy as jnp
from jax import lax
from jax.experimental import pallas as pl
from jax.experimental.pallas import tpu as pltpu
from jax.experimental.pallas import tpu_sc as plsc

VOCAB = 100000
EMB = 128
HID = 1024
TAGS = 256
B = 4096
L = 200

NC = 2
NS = 16
NW = NC * NS
RPW = B // NW
HALF = L // 2
NVEC = EMB // 16
INV_L = 1.0 / L


def _pool_body(x_hbm, emb_hbm, out_hbm, idx_all, rows_v, out_v, sems):
    wid = lax.axis_index("s") * NC + lax.axis_index("c")
    base = wid * RPW

    pltpu.sync_copy(x_hbm.at[pl.ds(base, RPW)], idx_all)

    def fire(row, buf):
        pltpu.async_copy(emb_hbm.at[idx_all.at[row, 0]],
                         rows_v.at[buf, pl.ds(0, HALF)], sems.at[buf])
        pltpu.async_copy(emb_hbm.at[idx_all.at[row, 1]],
                         rows_v.at[buf, pl.ds(HALF, HALF)], sems.at[buf])

    def drain(buf):
        pltpu.make_async_copy(emb_hbm.at[idx_all.at[0, 0]],
                              rows_v.at[buf, pl.ds(0, HALF)], sems.at[buf]).wait()
        pltpu.make_async_copy(emb_hbm.at[idx_all.at[0, 1]],
                              rows_v.at[buf, pl.ds(HALF, HALF)], sems.at[buf]).wait()

    fire(0, 0)

    def outer(i):
        for b in range(2):
            row = i + b
            fire(lax.min(row + 1, RPW - 1), 1 - b)
            drain(b)

            def red(j, accs):
                return tuple(a + rows_v[b, j, pl.ds(16 * k, 16)]
                             for k, a in enumerate(accs))

            accs = lax.fori_loop(
                0, L, red,
                tuple(jnp.zeros((16,), jnp.float32) for _ in range(NVEC)))
            for k in range(NVEC):
                out_v[row, pl.ds(16 * k, 16)] = accs[k] * INV_L

    pl.loop(0, RPW, step=2)(outer)
    drain(0)
    pltpu.sync_copy(out_v, out_hbm.at[pl.ds(base, RPW)])


@functools.partial(jax.jit, static_argnames=())
def _pool(x3, emb):
    mesh = plsc.VectorSubcoreMesh(core_axis_name="c", subcore_axis_name="s")
    return pl.kernel(
        _pool_body,
        out_type=jax.ShapeDtypeStruct((B, EMB), jnp.float32),
        mesh=mesh,
        scratch_types=[
            pltpu.VMEM((RPW, 2, HALF), jnp.int32),
            pltpu.VMEM((2, L, EMB), jnp.float32),
            pltpu.VMEM((RPW, EMB), jnp.float32),
            pltpu.SemaphoreType.DMA((2,)),
        ],
    )(x3, emb)


def _mlp_block(p_ref, w1_ref, b1_ref, w2_ref, b2_ref, o_ref):
    h = jnp.dot(p_ref[...], w1_ref[...], preferred_element_type=jnp.float32)
    h = jnp.maximum(h + b1_ref[...], 0.0)
    o_ref[...] = jnp.dot(h, w2_ref[...],
                         preferred_element_type=jnp.float32) + b2_ref[...]


def _mlp(pooled, W1, b1, W2, b2):
    BM = 512
    return pl.pallas_call(
        _mlp_block,
        grid=(B // BM,),
        in_specs=[
            pl.BlockSpec((BM, EMB), lambda i: (i, 0)),
            pl.BlockSpec((EMB, HID), lambda i: (0, 0)),
            pl.BlockSpec((1, HID), lambda i: (0, 0)),
            pl.BlockSpec((HID, TAGS), lambda i: (0, 0)),
            pl.BlockSpec((1, TAGS), lambda i: (0, 0)),
        ],
        out_specs=pl.BlockSpec((BM, TAGS), lambda i: (i, 0)),
        out_shape=jax.ShapeDtypeStruct((B, TAGS), jnp.float32),
    )(pooled, W1, b1.reshape(1, HID), W2, b2.reshape(1, TAGS))


def kernel(x, emb, W1, b1, W2, b2):
    x3 = x.astype(jnp.int32).reshape(B, 2, HALF)
    pooled = _pool(x3, emb)
    return _mlp(pooled, W1, b1, W2, b2)
